# baseline (speedup 1.0000x reference)
import functools

import jax
import jax.numpy as jnp
from jax import lax
from jax.experimental import pallas as pl
from jax.experimental.pallas import tpu as pltpu
from jax.experimental.pallas import tpu_sc as plsc

_NC = 2
_NS = 16
_NW = _NC * _NS
_LN = 16
_F32 = jnp.float32


def _sc_mesh():
    return plsc.VectorSubcoreMesh(core_axis_name="c", subcore_axis_name="s")


def _p1_body(a1h, a2h, a3e, src, dst, ehat, stats,
             sidx0, didx0, r10, r20, r30, ob0,
             sidx1, didx1, r11, r21, r31, ob1,
             ssum, ssq, statv, sin0, sin1, sout0, sout1,
             *, EW, B, H):
    c = lax.axis_index("c")
    s = lax.axis_index("s")
    wid = s * _NC + c
    e0 = wid * EW
    nch = EW // B
    HS = H // _LN
    z = jnp.zeros((_LN,), _F32)
    for j in range(HS):
        ssum[j] = z
        ssq[j] = z

    slots = (
        (sidx0, didx0, r10, r20, r30, ob0, sin0, sout0),
        (sidx1, didx1, r11, r21, r31, ob1, sin1, sout1),
    )

    def prefetch(k, sl):
        si, di, r1, r2, r3, _, sin, _ = sl
        base = e0 + k * B
        pltpu.sync_copy(src.at[pl.ds(base, B)], si)
        pltpu.sync_copy(dst.at[pl.ds(base, B)], di)
        pltpu.async_copy(a1h.at[si], r1, sin)
        pltpu.async_copy(a2h.at[di], r2, sin)
        pltpu.async_copy(a3e.at[pl.ds(base, B)], r3, sin)

    def process(k, sl, wait_out):
        si, di, r1, r2, r3, ob, sin, sout = sl
        pltpu.make_async_copy(a1h.at[si], r1, sin).wait()
        pltpu.make_async_copy(a2h.at[di], r2, sin).wait()
        pltpu.make_async_copy(a3e.at[pl.ds(0, B)], r3, sin).wait()
        if wait_out:
            pltpu.make_async_copy(ob, ehat.at[pl.ds(0, B)], sout).wait()

        @pl.loop(0, B, unroll=2)
        def _edge(i):
            for j in range(HS):
                slx = pl.ds(j * _LN, _LN)
                g = r1[i, slx] + r2[i, slx] + r3[i, slx]
                ob[i, slx] = g
                plsc.addupdate(ssum.at[j], g)
                plsc.addupdate(ssq.at[j], g * g)

        pltpu.async_copy(ob, ehat.at[pl.ds(e0 + k * B, B)], sout)

    prefetch(0, slots[0])
    prefetch(1, slots[1])
    process(0, slots[0], wait_out=False)
    prefetch(2, slots[0])
    process(1, slots[1], wait_out=False)

    @pl.loop(1, nch // 2)
    def _pair(t):
        k = 2 * t
        prefetch(k + 1, slots[1])
        process(k, slots[0], wait_out=True)
        pl.when(k + 2 <= nch - 1)(lambda: prefetch(k + 2, slots[0]))
        process(k + 1, slots[1], wait_out=True)

    if nch % 2 == 1:
        process(nch - 1, slots[0], wait_out=True)

    pltpu.make_async_copy(ob0, ehat.at[pl.ds(0, B)], sout0).wait()
    pltpu.make_async_copy(ob1, ehat.at[pl.ds(0, B)], sout1).wait()

    for r in range(8):
        for j in range(2 * HS):
            statv[r, pl.ds(j * _LN, _LN)] = z
    for j in range(HS):
        statv[0, pl.ds(j * _LN, _LN)] = ssum[j]
        statv[0, pl.ds(H + j * _LN, _LN)] = ssq[j]
    pltpu.sync_copy(statv, stats.at[pl.ds(wid * 8, 8)])


def _make_p1(N, E, H):
    EW = E // _NW
    B = 80
    assert EW % B == 0 and EW // B >= 4
    body = functools.partial(_p1_body, EW=EW, B=B, H=H)
    dbl = [
        pltpu.VMEM((B,), jnp.int32),
        pltpu.VMEM((B,), jnp.int32),
        pltpu.VMEM((B, H), _F32),
        pltpu.VMEM((B, H), _F32),
        pltpu.VMEM((B, H), _F32),
        pltpu.VMEM((B, H), _F32),
    ]
    return pl.kernel(
        body,
        out_type=[
            jax.ShapeDtypeStruct((E, H), _F32),
            jax.ShapeDtypeStruct((_NW * 8, 2 * H), _F32),
        ],
        mesh=_sc_mesh(),
        scratch_types=dbl + dbl + [
            pltpu.VMEM((H // _LN, _LN), _F32),
            pltpu.VMEM((H // _LN, _LN), _F32),
            pltpu.VMEM((8, 2 * H), _F32),
            pltpu.SemaphoreType.DMA,
            pltpu.SemaphoreType.DMA,
            pltpu.SemaphoreType.DMA,
            pltpu.SemaphoreType.DMA,
        ],
    )


def _p2_body(e_new, vh, src, dst, den_p, num_p,
             eidx0, sidx0, enb0, pay0, vrows0,
             eidx1, sidx1, enb1, pay1, vrows1,
             wbuf, acc, sin0, sin1, ssc0, ssc1,
             *, EW, B, H, N, WR):
    c = lax.axis_index("c")
    s = lax.axis_index("s")
    wid = s * _NC + c
    e0 = wid * EW
    nch = EW // B
    HS = H // _LN
    nrc = N // WR
    z = jnp.zeros((_LN,), _F32)

    slots = (
        (eidx0, sidx0, enb0, pay0, vrows0, sin0, ssc0),
        (eidx1, sidx1, enb1, pay1, vrows1, sin1, ssc1),
    )

    for d, outp in ((0, den_p), (1, num_p)):
        @pl.loop(0, WR)
        def _zb(i):
            for j in range(HS):
                wbuf[i, pl.ds(j * _LN, _LN)] = z

        @pl.loop(s, nrc, step=_NS)
        def _za(t):
            pltpu.sync_copy(wbuf, acc.at[pl.ds(t * WR, WR)])

        plsc.subcore_barrier()

        def prefetch(k, sl):
            _, si, enb, _, vrows, sin, _ = sl
            base = e0 + k * B
            pltpu.async_copy(e_new.at[pl.ds(base, B)], enb, sin)
            if d == 1:
                pltpu.sync_copy(src.at[pl.ds(base, B)], si)
                pltpu.async_copy(vh.at[si], vrows, sin)

        def process(k, sl, wait_sc):
            ei, si, enb, pay, vrows, sin, ssc = sl
            pltpu.make_async_copy(e_new.at[pl.ds(0, B)], enb, sin).wait()
            if d == 1:
                pltpu.make_async_copy(vh.at[si], vrows, sin).wait()
            if wait_sc:
                pltpu.make_async_copy(pay, acc.at[ei], ssc).wait()
            pltpu.sync_copy(dst.at[pl.ds(e0 + k * B, B)], ei)

            @pl.loop(0, B, unroll=2)
            def _edge(i):
                for j in range(HS):
                    sl2 = pl.ds(j * _LN, _LN)
                    xv = enb[i, sl2]
                    sg = 1.0 / (1.0 + jnp.exp(-xv))
                    if d == 1:
                        sg = sg * vrows[i, sl2]
                    pay[i, sl2] = sg

            pltpu.async_copy(pay, acc.at[ei], ssc, add=True)

        prefetch(0, slots[0])
        prefetch(1, slots[1])
        process(0, slots[0], wait_sc=False)
        prefetch(2, slots[0])
        process(1, slots[1], wait_sc=False)

        @pl.loop(1, nch // 2)
        def _pair(t):
            k = 2 * t
            prefetch(k + 1, slots[1])
            process(k, slots[0], wait_sc=True)
            pl.when(k + 2 <= nch - 1)(lambda: prefetch(k + 2, slots[0]))
            process(k + 1, slots[1], wait_sc=True)

        pltpu.make_async_copy(pay0, acc.at[eidx0], ssc0).wait()
        pltpu.make_async_copy(pay1, acc.at[eidx1], ssc1).wait()

        plsc.subcore_barrier()

        @pl.loop(s, nrc, step=_NS)
        def _wo(t):
            row = t * WR
            pltpu.sync_copy(acc.at[pl.ds(row, WR)], wbuf)
            pltpu.sync_copy(wbuf, outp.at[c, pl.ds(row, WR)])

        plsc.subcore_barrier()


def _make_p2(N, E, H):
    EW = E // _NW
    B = 40
    WR = 80
    assert N % WR == 0 and EW % B == 0 and (EW // B) % 2 == 0
    body = functools.partial(_p2_body, EW=EW, B=B, H=H, N=N, WR=WR)
    dbl = [
        pltpu.VMEM((B,), jnp.int32),
        pltpu.VMEM((B,), jnp.int32),
        pltpu.VMEM((B, H), _F32),
        pltpu.VMEM((B, H), _F32),
        pltpu.VMEM((B, H), _F32),
    ]
    return pl.kernel(
        body,
        out_type=[
            jax.ShapeDtypeStruct((_NC, N, H), _F32),
            jax.ShapeDtypeStruct((_NC, N, H), _F32),
        ],
        mesh=_sc_mesh(),
        scratch_types=dbl + dbl + [
            pltpu.VMEM((WR, H), _F32),
            pltpu.VMEM_SHARED((N, H), _F32),
            pltpu.SemaphoreType.DMA,
            pltpu.SemaphoreType.DMA,
            pltpu.SemaphoreType.DMA,
            pltpu.SemaphoreType.DMA,
        ],
    )


def _p3_body(p12m, p3m, src, dst, sv_out,
             sidx0, didx0, r10, r20, r30, ob0,
             sidx1, didx1, r11, r21, r31, ob1,
             sin0, sin1, sout0, sout1,
             *, EW, B, HS_DIM):
    c = lax.axis_index("c")
    s = lax.axis_index("s")
    wid = s * _NC + c
    e0 = wid * EW
    nch = EW // B
    JS = HS_DIM // _LN

    slots = (
        (sidx0, didx0, r10, r20, r30, ob0, sin0, sout0),
        (sidx1, didx1, r11, r21, r31, ob1, sin1, sout1),
    )

    def prefetch(k, sl):
        si, di, r1, r2, r3, _, sin, _ = sl
        base = e0 + k * B
        pltpu.sync_copy(src.at[pl.ds(base, B)], si)
        pltpu.sync_copy(dst.at[pl.ds(base, B)], di)
        pltpu.async_copy(p12m.at[si], r1, sin)
        pltpu.async_copy(p12m.at[di], r2, sin)
        pltpu.async_copy(p3m.at[pl.ds(base, B)], r3, sin)

    def process(k, sl, wait_out):
        si, di, r1, r2, r3, ob, sin, sout = sl
        pltpu.make_async_copy(p12m.at[si], r1, sin).wait()
        pltpu.make_async_copy(p12m.at[di], r2, sin).wait()
        pltpu.make_async_copy(p3m.at[pl.ds(0, B)], r3, sin).wait()
        if wait_out:
            pltpu.make_async_copy(ob, sv_out.at[pl.ds(0, B)], sout).wait()

        @pl.loop(0, B)
        def _edge(i):
            for j4 in range(JS):
                slx = pl.ds(j4 * _LN, _LN)
                v = (r1[i, slx] + r2[i, pl.ds(HS_DIM + j4 * _LN, _LN)]
                     + r3[i, slx])
                ob[i, slx] = jnp.maximum(v, 0.0)

        pltpu.async_copy(ob, sv_out.at[pl.ds(e0 + k * B, B)], sout)

    prefetch(0, slots[0])
    prefetch(1, slots[1])
    process(0, slots[0], wait_out=False)
    prefetch(2, slots[0])
    process(1, slots[1], wait_out=False)

    @pl.loop(1, nch // 2)
    def _pair(t):
        k = 2 * t
        prefetch(k + 1, slots[1])
        process(k, slots[0], wait_out=True)
        pl.when(k + 2 <= nch - 1)(lambda: prefetch(k + 2, slots[0]))
        process(k + 1, slots[1], wait_out=True)

    if nch % 2 == 1:
        process(nch - 1, slots[0], wait_out=True)

    pltpu.make_async_copy(ob0, sv_out.at[pl.ds(0, B)], sout0).wait()
    pltpu.make_async_copy(ob1, sv_out.at[pl.ds(0, B)], sout1).wait()


def _make_p3(E, HS_DIM):
    EW = E // _NW
    B = 80
    assert EW % B == 0 and EW // B >= 4
    body = functools.partial(_p3_body, EW=EW, B=B, HS_DIM=HS_DIM)
    dbl = [
        pltpu.VMEM((B,), jnp.int32),
        pltpu.VMEM((B,), jnp.int32),
        pltpu.VMEM((B, 2 * HS_DIM), _F32),
        pltpu.VMEM((B, 2 * HS_DIM), _F32),
        pltpu.VMEM((B, HS_DIM), _F32),
        pltpu.VMEM((B, HS_DIM), _F32),
    ]
    return pl.kernel(
        body,
        out_type=jax.ShapeDtypeStruct((E, HS_DIM), _F32),
        mesh=_sc_mesh(),
        scratch_types=dbl + dbl + [
            pltpu.SemaphoreType.DMA,
            pltpu.SemaphoreType.DMA,
            pltpu.SemaphoreType.DMA,
            pltpu.SemaphoreType.DMA,
        ],
    )


def _pe_body(pe_ref, wt_ref, b_ref, o_ref):
    o_ref[...] = (
        jnp.dot(pe_ref[...], wt_ref[...], preferred_element_type=_F32)
        + b_ref[...]
    )


def _edge_enc_body(e_ref, w1t, b1, w2t, b2, w3t, b3, o_ref, o_a3):
    t = jnp.dot(e_ref[...], w1t[...], preferred_element_type=_F32) + b1[...]
    t = jnp.maximum(t, 0.0)
    ee = jnp.dot(t, w2t[...], preferred_element_type=_F32) + b2[...]
    o_ref[...] = ee
    o_a3[...] = jnp.dot(ee, w3t[...], preferred_element_type=_F32) + b3[...]


def _node_mm_body(h_ref, w1t, b1, w2t, b2, w3t, b3, w4t, b4, o1, o2, o3, o4):
    h = h_ref[...]
    o1[...] = jnp.dot(h, w1t[...], preferred_element_type=_F32) + b1[...]
    o2[...] = jnp.dot(h, w2t[...], preferred_element_type=_F32) + b2[...]
    o3[...] = jnp.dot(h, w3t[...], preferred_element_type=_F32) + b3[...]
    o4[...] = jnp.dot(h, w4t[...], preferred_element_type=_F32) + b4[...]


def _mm_bias_body(x_ref, wt_ref, b_ref, o_ref):
    o_ref[...] = (
        jnp.dot(x_ref[...], wt_ref[...], preferred_element_type=_F32)
        + b_ref[...]
    )


def _bn_e_body(ehat, ee, stats, g, b, wt, bt, o, o_nxt, *, E, H):
    st = stats[...]
    ssum = jnp.sum(st[:, :H], axis=0)
    ssq = jnp.sum(st[:, H:], axis=0)
    mean = ssum * (1.0 / E)
    var = ssq * (1.0 / E) - mean * mean
    rstd = lax.rsqrt(var + 1e-5)
    xh = (ehat[...] - mean[None, :]) * rstd[None, :] * g[...] + b[...]
    en = ee[...] + jnp.maximum(xh, 0.0)
    o[...] = en
    o_nxt[...] = jnp.dot(en, wt[...], preferred_element_type=_F32) + bt[...]


def _h_upd_body(h, uh, denp, nump, g, b, o):
    den = denp[0] + denp[1] + 1e-6
    num = nump[0] + nump[1]
    t = uh[...] + num / den
    mean = jnp.mean(t, axis=0, keepdims=True)
    var = jnp.mean((t - mean) ** 2, axis=0, keepdims=True)
    xh = (t - mean) * lax.rsqrt(var + 1e-5) * g[...] + b[...]
    o[...] = h[...] + jnp.maximum(xh, 0.0)


def _full_spec(shape):
    return pl.BlockSpec(shape, lambda i: (0,) * len(shape))


def _row_spec(be, cols):
    return pl.BlockSpec((be, cols), lambda i: (i, 0))


def kernel(x, e, pe, edge_index, params):
    p = params
    N, H = x.shape
    E, D_EDGE = e.shape
    PE_DIM = pe.shape[1]
    H_EDGE = p["e1_w"].shape[0]
    HS_DIM = p["s1_w"].shape[0]
    L = p["A1_w"].shape[0]
    assert E % _NW == 0 and (E // _NW) % 80 == 0 and N % _NS == 0

    src = edge_index[0]
    dst = edge_index[1]

    BE = 4000
    grid_e = (E // BE,)

    h = pl.pallas_call(
        _pe_body,
        out_shape=jax.ShapeDtypeStruct((N, H), _F32),
    )(pe, p["pe_w"].T, p["pe_b"][None, :])

    ee, a3e = pl.pallas_call(
        _edge_enc_body,
        grid=grid_e,
        in_specs=[
            _row_spec(BE, D_EDGE),
            _full_spec((D_EDGE, H_EDGE)),
            _full_spec((1, H_EDGE)),
            _full_spec((H_EDGE, H)),
            _full_spec((1, H)),
            _full_spec((H, H)),
            _full_spec((1, H)),
        ],
        out_specs=[_row_spec(BE, H), _row_spec(BE, H)],
        out_shape=[jax.ShapeDtypeStruct((E, H), _F32)] * 2,
    )(e, p["e1_w"].T, p["e1_b"][None, :], p["e2_w"].T, p["e2_b"][None, :],
      p["A3_w"][0].T, p["A3_b"][0][None, :])

    p1_fn = _make_p1(N, E, H)
    p2_fn = _make_p2(N, E, H)
    s1 = p["s1_w"]

    for l in range(L):
        a1h, a2h, uh, vh = pl.pallas_call(
            _node_mm_body,
            out_shape=[jax.ShapeDtypeStruct((N, H), _F32)] * 4,
        )(
            h,
            p["A1_w"][l].T, p["A1_b"][l][None, :],
            p["A2_w"][l].T, p["A2_b"][l][None, :],
            p["U_w"][l].T, p["U_b"][l][None, :],
            p["V_w"][l].T, p["V_b"][l][None, :],
        )

        ehat, stats = p1_fn(a1h, a2h, a3e, src, dst)

        if l < L - 1:
            wt, bt = p["A3_w"][l + 1].T, p["A3_b"][l + 1][None, :]
        else:
            wt, bt = s1[:, 2 * H:].T, p["s1_b"][None, :]
        nxt_w = wt.shape[1]
        e_new, nxt = pl.pallas_call(
            functools.partial(_bn_e_body, E=E, H=H),
            grid=grid_e,
            in_specs=[
                _row_spec(BE, H),
                _row_spec(BE, H),
                _full_spec((_NW * 8, 2 * H)),
                _full_spec((1, H)),
                _full_spec((1, H)),
                _full_spec((H, nxt_w)),
                _full_spec((1, nxt_w)),
            ],
            out_specs=[_row_spec(BE, H), _row_spec(BE, nxt_w)],
            out_shape=[
                jax.ShapeDtypeStruct((E, H), _F32),
                jax.ShapeDtypeStruct((E, nxt_w), _F32),
            ],
        )(ehat, ee, stats, p["bn_e_g"][l][None, :], p["bn_e_b"][l][None, :],
          wt, bt)

        den_p, num_p = p2_fn(e_new, vh, src, dst)

        h = pl.pallas_call(
            _h_upd_body,
            out_shape=jax.ShapeDtypeStruct((N, H), _F32),
        )(h, uh, den_p, num_p, p["bn_h_g"][l][None, :], p["bn_h_b"][l][None, :])

        ee = e_new
        a3e = nxt

    p3s = a3e
    wcat = jnp.concatenate([s1[:, :H].T, s1[:, H:2 * H].T], axis=1)
    p12 = pl.pallas_call(
        _mm_bias_body,
        out_shape=jax.ShapeDtypeStruct((N, 2 * HS_DIM), _F32),
    )(h, wcat, jnp.zeros((1, 2 * HS_DIM), _F32))

    sv = _make_p3(E, HS_DIM)(p12, p3s, src, dst)

    scores = pl.pallas_call(
        _mm_bias_body,
        grid=grid_e,
        in_specs=[
            _row_spec(BE, HS_DIM),
            _full_spec((HS_DIM, 1)),
            _full_spec((1, 1)),
        ],
        out_specs=_row_spec(BE, 1),
        out_shape=jax.ShapeDtypeStruct((E, 1), _F32),
    )(sv, p["s2_w"].T, p["s2_b"][None, :])
    return scores


# speedup vs baseline: 2.6318x; 2.6318x over previous
import functools

import jax
import jax.numpy as jnp
from jax import lax
from jax.experimental import pallas as pl
from jax.experimental.pallas import tpu as pltpu
from jax.experimental.pallas import tpu_sc as plsc

_NC = 2
_NS = 16
_NW = _NC * _NS
_LN = 16
_F32 = jnp.float32


def _sc_mesh():
    return plsc.VectorSubcoreMesh(core_axis_name="c", subcore_axis_name="s")


def _p1_body(a1h, a2h, a3e, src, dst, ehat, stats,
             sidx0, didx0, r10, r20, r30, ob0,
             sidx1, didx1, r11, r21, r31, ob1,
             ssum, ssq, statv, sin0, sin1, sout0, sout1,
             *, EW, B, H):
    c = lax.axis_index("c")
    s = lax.axis_index("s")
    wid = s * _NC + c
    e0 = wid * EW
    nch = EW // B
    HS = H // _LN
    z = jnp.zeros((_LN,), _F32)
    for j in range(HS):
        ssum[j] = z
        ssq[j] = z

    slots = (
        (sidx0, didx0, r10, r20, r30, ob0, sin0, sout0),
        (sidx1, didx1, r11, r21, r31, ob1, sin1, sout1),
    )

    def prefetch(k, sl):
        si, di, r1, r2, r3, _, sin, _ = sl
        base = e0 + k * B
        pltpu.sync_copy(src.at[pl.ds(base, B)], si)
        pltpu.sync_copy(dst.at[pl.ds(base, B)], di)
        pltpu.async_copy(a1h.at[si], r1, sin)
        pltpu.async_copy(a2h.at[di], r2, sin)
        pltpu.async_copy(a3e.at[pl.ds(base, B)], r3, sin)

    def process(k, sl, wait_out):
        si, di, r1, r2, r3, ob, sin, sout = sl
        pltpu.make_async_copy(a1h.at[si], r1, sin).wait()
        pltpu.make_async_copy(a2h.at[di], r2, sin).wait()
        pltpu.make_async_copy(a3e.at[pl.ds(0, B)], r3, sin).wait()
        if wait_out:
            pltpu.make_async_copy(ob, ehat.at[pl.ds(0, B)], sout).wait()

        @pl.loop(0, B)
        def _edge(i):
            for j in range(HS):
                slx = pl.ds(j * _LN, _LN)
                g = r1[i, slx] + r2[i, slx] + r3[i, slx]
                ob[i, slx] = g
                plsc.addupdate(ssum.at[j], g)
                plsc.addupdate(ssq.at[j], g * g)

        pltpu.async_copy(ob, ehat.at[pl.ds(e0 + k * B, B)], sout)

    prefetch(0, slots[0])
    prefetch(1, slots[1])
    process(0, slots[0], wait_out=False)
    prefetch(2, slots[0])
    process(1, slots[1], wait_out=False)

    @pl.loop(1, nch // 2)
    def _pair(t):
        k = 2 * t
        prefetch(k + 1, slots[1])
        process(k, slots[0], wait_out=True)
        pl.when(k + 2 <= nch - 1)(lambda: prefetch(k + 2, slots[0]))
        process(k + 1, slots[1], wait_out=True)

    if nch % 2 == 1:
        process(nch - 1, slots[0], wait_out=True)

    pltpu.make_async_copy(ob0, ehat.at[pl.ds(0, B)], sout0).wait()
    pltpu.make_async_copy(ob1, ehat.at[pl.ds(0, B)], sout1).wait()

    for r in range(8):
        for j in range(2 * HS):
            statv[r, pl.ds(j * _LN, _LN)] = z
    for j in range(HS):
        statv[0, pl.ds(j * _LN, _LN)] = ssum[j]
        statv[0, pl.ds(H + j * _LN, _LN)] = ssq[j]
    pltpu.sync_copy(statv, stats.at[pl.ds(wid * 8, 8)])


def _make_p1(N, E, H):
    EW = E // _NW
    B = 80
    assert EW % B == 0 and EW // B >= 4
    body = functools.partial(_p1_body, EW=EW, B=B, H=H)
    dbl = [
        pltpu.VMEM((B,), jnp.int32),
        pltpu.VMEM((B,), jnp.int32),
        pltpu.VMEM((B, H), _F32),
        pltpu.VMEM((B, H), _F32),
        pltpu.VMEM((B, H), _F32),
        pltpu.VMEM((B, H), _F32),
    ]
    return pl.kernel(
        body,
        out_type=[
            jax.ShapeDtypeStruct((E, H), _F32),
            jax.ShapeDtypeStruct((_NW * 8, 2 * H), _F32),
        ],
        mesh=_sc_mesh(),
        scratch_types=dbl + dbl + [
            pltpu.VMEM((H // _LN, _LN), _F32),
            pltpu.VMEM((H // _LN, _LN), _F32),
            pltpu.VMEM((8, 2 * H), _F32),
            pltpu.SemaphoreType.DMA,
            pltpu.SemaphoreType.DMA,
            pltpu.SemaphoreType.DMA,
            pltpu.SemaphoreType.DMA,
        ],
    )


def _p2_body(e_new, vh, src, dst, den_p, num_p,
             eidx0, sidx0, enb0, pay0, vrows0,
             eidx1, sidx1, enb1, pay1, vrows1,
             wbuf, acc, sin0, sin1, ssc0, ssc1,
             *, EW, B, H, N, WR):
    c = lax.axis_index("c")
    s = lax.axis_index("s")
    wid = s * _NC + c
    e0 = wid * EW
    nch = EW // B
    HS = H // _LN
    nrc = N // WR
    z = jnp.zeros((_LN,), _F32)

    slots = (
        (eidx0, sidx0, enb0, pay0, vrows0, sin0, ssc0),
        (eidx1, sidx1, enb1, pay1, vrows1, sin1, ssc1),
    )

    for d, outp in ((0, den_p), (1, num_p)):
        @pl.loop(0, WR)
        def _zb(i):
            for j in range(HS):
                wbuf[i, pl.ds(j * _LN, _LN)] = z

        @pl.loop(s, nrc, step=_NS)
        def _za(t):
            pltpu.sync_copy(wbuf, acc.at[pl.ds(t * WR, WR)])

        plsc.subcore_barrier()

        def prefetch(k, sl):
            _, si, enb, _, vrows, sin, _ = sl
            base = e0 + k * B
            pltpu.async_copy(e_new.at[pl.ds(base, B)], enb, sin)
            if d == 1:
                pltpu.sync_copy(src.at[pl.ds(base, B)], si)
                pltpu.async_copy(vh.at[si], vrows, sin)

        def process(k, sl, wait_sc):
            ei, si, enb, pay, vrows, sin, ssc = sl
            pltpu.make_async_copy(e_new.at[pl.ds(0, B)], enb, sin).wait()
            if d == 1:
                pltpu.make_async_copy(vh.at[si], vrows, sin).wait()
            if wait_sc:
                pltpu.make_async_copy(pay, acc.at[ei], ssc).wait()
            pltpu.sync_copy(dst.at[pl.ds(e0 + k * B, B)], ei)

            @pl.loop(0, B)
            def _edge(i):
                for j in range(HS):
                    sl2 = pl.ds(j * _LN, _LN)
                    xv = enb[i, sl2]
                    sg = 1.0 / (1.0 + jnp.exp(-xv))
                    if d == 1:
                        sg = sg * vrows[i, sl2]
                    pay[i, sl2] = sg

            pltpu.async_copy(pay, acc.at[ei], ssc, add=True)

        prefetch(0, slots[0])
        prefetch(1, slots[1])
        process(0, slots[0], wait_sc=False)
        prefetch(2, slots[0])
        process(1, slots[1], wait_sc=False)

        @pl.loop(1, nch // 2)
        def _pair(t):
            k = 2 * t
            prefetch(k + 1, slots[1])
            process(k, slots[0], wait_sc=True)
            pl.when(k + 2 <= nch - 1)(lambda: prefetch(k + 2, slots[0]))
            process(k + 1, slots[1], wait_sc=True)

        pltpu.make_async_copy(pay0, acc.at[eidx0], ssc0).wait()
        pltpu.make_async_copy(pay1, acc.at[eidx1], ssc1).wait()

        plsc.subcore_barrier()

        @pl.loop(s, nrc, step=_NS)
        def _wo(t):
            row = t * WR
            pltpu.sync_copy(acc.at[pl.ds(row, WR)], wbuf)
            pltpu.sync_copy(wbuf, outp.at[c, pl.ds(row, WR)])

        plsc.subcore_barrier()


def _make_p2(N, E, H):
    EW = E // _NW
    B = 40
    WR = 80
    assert N % WR == 0 and EW % B == 0 and (EW // B) % 2 == 0
    body = functools.partial(_p2_body, EW=EW, B=B, H=H, N=N, WR=WR)
    dbl = [
        pltpu.VMEM((B,), jnp.int32),
        pltpu.VMEM((B,), jnp.int32),
        pltpu.VMEM((B, H), _F32),
        pltpu.VMEM((B, H), _F32),
        pltpu.VMEM((B, H), _F32),
    ]
    return pl.kernel(
        body,
        out_type=[
            jax.ShapeDtypeStruct((_NC, N, H), _F32),
            jax.ShapeDtypeStruct((_NC, N, H), _F32),
        ],
        mesh=_sc_mesh(),
        scratch_types=dbl + dbl + [
            pltpu.VMEM((WR, H), _F32),
            pltpu.VMEM_SHARED((N, H), _F32),
            pltpu.SemaphoreType.DMA,
            pltpu.SemaphoreType.DMA,
            pltpu.SemaphoreType.DMA,
            pltpu.SemaphoreType.DMA,
        ],
    )


def _p3_body(p12m, p3m, src, dst, w2, sv_out,
             sidx0, didx0, r10, r20, r30, ob0,
             sidx1, didx1, r11, r21, r31, ob1,
             w2v, sin0, sin1, sout0, sout1,
             *, EW, B, HS_DIM):
    c = lax.axis_index("c")
    s = lax.axis_index("s")
    wid = s * _NC + c
    e0 = wid * EW
    nch = EW // B
    JS = HS_DIM // _LN

    slots = (
        (sidx0, didx0, r10, r20, r30, ob0, sin0, sout0),
        (sidx1, didx1, r11, r21, r31, ob1, sin1, sout1),
    )

    pltpu.sync_copy(w2, w2v)
    wvecs = [w2v[pl.ds(q * _LN, _LN)] for q in range(JS)]

    def prefetch(k, sl):
        si, di, r1, r2, r3, _, sin, _ = sl
        base = e0 + k * B
        pltpu.sync_copy(src.at[pl.ds(base, B)], si)
        pltpu.sync_copy(dst.at[pl.ds(base, B)], di)
        pltpu.async_copy(p12m.at[si], r1, sin)
        pltpu.async_copy(p12m.at[di], r2, sin)
        pltpu.async_copy(p3m.at[pl.ds(base, B)], r3, sin)

    def process(k, sl, wait_out):
        si, di, r1, r2, r3, ob, sin, sout = sl
        pltpu.make_async_copy(p12m.at[si], r1, sin).wait()
        pltpu.make_async_copy(p12m.at[di], r2, sin).wait()
        pltpu.make_async_copy(p3m.at[pl.ds(0, B)], r3, sin).wait()
        if wait_out:
            pltpu.make_async_copy(ob, sv_out.at[pl.ds(0, B)], sout).wait()

        @pl.loop(0, B)
        def _edge(i):
            acc = jnp.zeros((_LN,), _F32)
            for j4 in range(JS):
                slx = pl.ds(j4 * _LN, _LN)
                v = (r1[i, slx] + r2[i, pl.ds(HS_DIM + j4 * _LN, _LN)]
                     + r3[i, slx])
                acc = acc + jnp.maximum(v, 0.0) * wvecs[j4]
            ob[i, pl.ds(0, _LN)] = acc

        pltpu.async_copy(ob, sv_out.at[pl.ds(e0 + k * B, B)], sout)

    prefetch(0, slots[0])
    prefetch(1, slots[1])
    process(0, slots[0], wait_out=False)
    prefetch(2, slots[0])
    process(1, slots[1], wait_out=False)

    @pl.loop(1, nch // 2)
    def _pair(t):
        k = 2 * t
        prefetch(k + 1, slots[1])
        process(k, slots[0], wait_out=True)
        pl.when(k + 2 <= nch - 1)(lambda: prefetch(k + 2, slots[0]))
        process(k + 1, slots[1], wait_out=True)

    if nch % 2 == 1:
        process(nch - 1, slots[0], wait_out=True)

    pltpu.make_async_copy(ob0, sv_out.at[pl.ds(0, B)], sout0).wait()
    pltpu.make_async_copy(ob1, sv_out.at[pl.ds(0, B)], sout1).wait()


def _make_p3(E, HS_DIM):
    EW = E // _NW
    B = 80
    assert EW % B == 0 and EW // B >= 4
    body = functools.partial(_p3_body, EW=EW, B=B, HS_DIM=HS_DIM)
    dbl = [
        pltpu.VMEM((B,), jnp.int32),
        pltpu.VMEM((B,), jnp.int32),
        pltpu.VMEM((B, 2 * HS_DIM), _F32),
        pltpu.VMEM((B, 2 * HS_DIM), _F32),
        pltpu.VMEM((B, HS_DIM), _F32),
        pltpu.VMEM((B, _LN), _F32),
    ]
    return pl.kernel(
        body,
        out_type=jax.ShapeDtypeStruct((E, _LN), _F32),
        mesh=_sc_mesh(),
        scratch_types=dbl + dbl + [
            pltpu.VMEM((HS_DIM,), _F32),
            pltpu.SemaphoreType.DMA,
            pltpu.SemaphoreType.DMA,
            pltpu.SemaphoreType.DMA,
            pltpu.SemaphoreType.DMA,
        ],
    )


def _pe_body(pe_ref, wt_ref, b_ref, o_ref):
    o_ref[...] = (
        jnp.dot(pe_ref[...], wt_ref[...], preferred_element_type=_F32)
        + b_ref[...]
    )


def _edge_enc_body(e_ref, w1t, b1, w2t, b2, w3t, b3, o_ref, o_a3):
    t = jnp.dot(e_ref[...], w1t[...], preferred_element_type=_F32) + b1[...]
    t = jnp.maximum(t, 0.0)
    ee = jnp.dot(t, w2t[...], preferred_element_type=_F32) + b2[...]
    o_ref[...] = ee
    o_a3[...] = jnp.dot(ee, w3t[...], preferred_element_type=_F32) + b3[...]


def _node_mm_body(h_ref, w1t, b1, w2t, b2, w3t, b3, w4t, b4, o1, o2, o3, o4):
    h = h_ref[...]
    o1[...] = jnp.dot(h, w1t[...], preferred_element_type=_F32) + b1[...]
    o2[...] = jnp.dot(h, w2t[...], preferred_element_type=_F32) + b2[...]
    o3[...] = jnp.dot(h, w3t[...], preferred_element_type=_F32) + b3[...]
    o4[...] = jnp.dot(h, w4t[...], preferred_element_type=_F32) + b4[...]


def _mm_bias_body(x_ref, wt_ref, b_ref, o_ref):
    o_ref[...] = (
        jnp.dot(x_ref[...], wt_ref[...], preferred_element_type=_F32)
        + b_ref[...]
    )


def _bn_e_body(ehat, ee, stats, g, b, wt, bt, o, o_nxt, *, E, H):
    st = stats[...]
    ssum = jnp.sum(st[:, :H], axis=0)
    ssq = jnp.sum(st[:, H:], axis=0)
    mean = ssum * (1.0 / E)
    var = ssq * (1.0 / E) - mean * mean
    rstd = lax.rsqrt(var + 1e-5)
    xh = (ehat[...] - mean[None, :]) * rstd[None, :] * g[...] + b[...]
    en = ee[...] + jnp.maximum(xh, 0.0)
    o[...] = en
    o_nxt[...] = jnp.dot(en, wt[...], preferred_element_type=_F32) + bt[...]


def _score_sum_body(sv, b2, o):
    o[...] = jnp.sum(sv[...], axis=1, keepdims=True) + b2[...]


def _h_upd_body(h, uh, denp, nump, g, b, o):
    den = denp[0] + denp[1] + 1e-6
    num = nump[0] + nump[1]
    t = uh[...] + num / den
    mean = jnp.mean(t, axis=0, keepdims=True)
    var = jnp.mean((t - mean) ** 2, axis=0, keepdims=True)
    xh = (t - mean) * lax.rsqrt(var + 1e-5) * g[...] + b[...]
    o[...] = h[...] + jnp.maximum(xh, 0.0)


def _full_spec(shape):
    return pl.BlockSpec(shape, lambda i: (0,) * len(shape))


def _row_spec(be, cols):
    return pl.BlockSpec((be, cols), lambda i: (i, 0))


def kernel(x, e, pe, edge_index, params):
    p = params
    N, H = x.shape
    E, D_EDGE = e.shape
    PE_DIM = pe.shape[1]
    H_EDGE = p["e1_w"].shape[0]
    HS_DIM = p["s1_w"].shape[0]
    L = p["A1_w"].shape[0]
    assert E % _NW == 0 and (E // _NW) % 80 == 0 and N % _NS == 0

    src = edge_index[0]
    dst = edge_index[1]

    BE = 4000
    grid_e = (E // BE,)

    h = pl.pallas_call(
        _pe_body,
        out_shape=jax.ShapeDtypeStruct((N, H), _F32),
    )(pe, p["pe_w"].T, p["pe_b"][None, :])

    ee, a3e = pl.pallas_call(
        _edge_enc_body,
        grid=grid_e,
        in_specs=[
            _row_spec(BE, D_EDGE),
            _full_spec((D_EDGE, H_EDGE)),
            _full_spec((1, H_EDGE)),
            _full_spec((H_EDGE, H)),
            _full_spec((1, H)),
            _full_spec((H, H)),
            _full_spec((1, H)),
        ],
        out_specs=[_row_spec(BE, H), _row_spec(BE, H)],
        out_shape=[jax.ShapeDtypeStruct((E, H), _F32)] * 2,
    )(e, p["e1_w"].T, p["e1_b"][None, :], p["e2_w"].T, p["e2_b"][None, :],
      p["A3_w"][0].T, p["A3_b"][0][None, :])

    p1_fn = _make_p1(N, E, H)
    p2_fn = _make_p2(N, E, H)
    s1 = p["s1_w"]

    for l in range(L):
        a1h, a2h, uh, vh = pl.pallas_call(
            _node_mm_body,
            out_shape=[jax.ShapeDtypeStruct((N, H), _F32)] * 4,
        )(
            h,
            p["A1_w"][l].T, p["A1_b"][l][None, :],
            p["A2_w"][l].T, p["A2_b"][l][None, :],
            p["U_w"][l].T, p["U_b"][l][None, :],
            p["V_w"][l].T, p["V_b"][l][None, :],
        )

        ehat, stats = p1_fn(a1h, a2h, a3e, src, dst)

        if l < L - 1:
            wt, bt = p["A3_w"][l + 1].T, p["A3_b"][l + 1][None, :]
        else:
            wt, bt = s1[:, 2 * H:].T, p["s1_b"][None, :]
        nxt_w = wt.shape[1]
        e_new, nxt = pl.pallas_call(
            functools.partial(_bn_e_body, E=E, H=H),
            grid=grid_e,
            in_specs=[
                _row_spec(BE, H),
                _row_spec(BE, H),
                _full_spec((_NW * 8, 2 * H)),
                _full_spec((1, H)),
                _full_spec((1, H)),
                _full_spec((H, nxt_w)),
                _full_spec((1, nxt_w)),
            ],
            out_specs=[_row_spec(BE, H), _row_spec(BE, nxt_w)],
            out_shape=[
                jax.ShapeDtypeStruct((E, H), _F32),
                jax.ShapeDtypeStruct((E, nxt_w), _F32),
            ],
        )(ehat, ee, stats, p["bn_e_g"][l][None, :], p["bn_e_b"][l][None, :],
          wt, bt)

        den_p, num_p = p2_fn(e_new, vh, src, dst)

        h = pl.pallas_call(
            _h_upd_body,
            out_shape=jax.ShapeDtypeStruct((N, H), _F32),
        )(h, uh, den_p, num_p, p["bn_h_g"][l][None, :], p["bn_h_b"][l][None, :])

        ee = e_new
        a3e = nxt

    p3s = a3e
    wcat = jnp.concatenate([s1[:, :H].T, s1[:, H:2 * H].T], axis=1)
    p12 = pl.pallas_call(
        _mm_bias_body,
        out_shape=jax.ShapeDtypeStruct((N, 2 * HS_DIM), _F32),
    )(h, wcat, jnp.zeros((1, 2 * HS_DIM), _F32))

    sv = _make_p3(E, HS_DIM)(p12, p3s, src, dst, p["s2_w"][0])

    scores = pl.pallas_call(
        _score_sum_body,
        grid=grid_e,
        in_specs=[
            _row_spec(BE, _LN),
            _full_spec((1, 1)),
        ],
        out_specs=_row_spec(BE, 1),
        out_shape=jax.ShapeDtypeStruct((E, 1), _F32),
    )(sv, p["s2_b"][None, :])
    return scores


# speedup vs baseline: 2.6370x; 1.0020x over previous
import functools

import jax
import jax.numpy as jnp
from jax import lax
from jax.experimental import pallas as pl
from jax.experimental.pallas import tpu as pltpu
from jax.experimental.pallas import tpu_sc as plsc

_NC = 2
_NS = 16
_NW = _NC * _NS
_LN = 16
_F32 = jnp.float32


def _sc_mesh():
    return plsc.VectorSubcoreMesh(core_axis_name="c", subcore_axis_name="s")


def _p1_body(a1h, a2h, a3e, src, dst, ehat, stats,
             sidx0, didx0, r10, r20, r30, ob0,
             sidx1, didx1, r11, r21, r31, ob1,
             ssum, ssq, statv, sin0, sin1, sout0, sout1,
             *, EW, B, H):
    c = lax.axis_index("c")
    s = lax.axis_index("s")
    wid = s * _NC + c
    e0 = wid * EW
    nch = EW // B
    HS = H // _LN
    z = jnp.zeros((_LN,), _F32)
    for j in range(HS):
        ssum[j] = z
        ssq[j] = z

    slots = (
        (sidx0, didx0, r10, r20, r30, ob0, sin0, sout0),
        (sidx1, didx1, r11, r21, r31, ob1, sin1, sout1),
    )

    def prefetch(k, sl):
        si, di, r1, r2, r3, _, sin, _ = sl
        base = e0 + k * B
        pltpu.sync_copy(src.at[pl.ds(base, B)], si)
        pltpu.sync_copy(dst.at[pl.ds(base, B)], di)
        pltpu.async_copy(a1h.at[si], r1, sin)
        pltpu.async_copy(a2h.at[di], r2, sin)
        pltpu.async_copy(a3e.at[pl.ds(base, B)], r3, sin)

    def process(k, sl, wait_out):
        si, di, r1, r2, r3, ob, sin, sout = sl
        pltpu.make_async_copy(a1h.at[si], r1, sin).wait()
        pltpu.make_async_copy(a2h.at[di], r2, sin).wait()
        pltpu.make_async_copy(a3e.at[pl.ds(0, B)], r3, sin).wait()
        if wait_out:
            pltpu.make_async_copy(ob, ehat.at[pl.ds(0, B)], sout).wait()

        @pl.loop(0, B)
        def _edge(i):
            for j in range(HS):
                slx = pl.ds(j * _LN, _LN)
                g = r1[i, slx] + r2[i, slx] + r3[i, slx]
                ob[i, slx] = g
                plsc.addupdate(ssum.at[j], g)
                plsc.addupdate(ssq.at[j], g * g)

        pltpu.async_copy(ob, ehat.at[pl.ds(e0 + k * B, B)], sout)

    prefetch(0, slots[0])
    prefetch(1, slots[1])
    process(0, slots[0], wait_out=False)
    prefetch(2, slots[0])
    process(1, slots[1], wait_out=False)

    @pl.loop(1, nch // 2)
    def _pair(t):
        k = 2 * t
        prefetch(k + 1, slots[1])
        process(k, slots[0], wait_out=True)
        pl.when(k + 2 <= nch - 1)(lambda: prefetch(k + 2, slots[0]))
        process(k + 1, slots[1], wait_out=True)

    if nch % 2 == 1:
        process(nch - 1, slots[0], wait_out=True)

    pltpu.make_async_copy(ob0, ehat.at[pl.ds(0, B)], sout0).wait()
    pltpu.make_async_copy(ob1, ehat.at[pl.ds(0, B)], sout1).wait()

    for r in range(8):
        for j in range(2 * HS):
            statv[r, pl.ds(j * _LN, _LN)] = z
    for j in range(HS):
        statv[0, pl.ds(j * _LN, _LN)] = ssum[j]
        statv[0, pl.ds(H + j * _LN, _LN)] = ssq[j]
    pltpu.sync_copy(statv, stats.at[pl.ds(wid * 8, 8)])


def _make_p1(N, E, H):
    EW = E // _NW
    B = 80
    assert EW % B == 0 and EW // B >= 4
    body = functools.partial(_p1_body, EW=EW, B=B, H=H)
    dbl = [
        pltpu.VMEM((B,), jnp.int32),
        pltpu.VMEM((B,), jnp.int32),
        pltpu.VMEM((B, H), _F32),
        pltpu.VMEM((B, H), _F32),
        pltpu.VMEM((B, H), _F32),
        pltpu.VMEM((B, H), _F32),
    ]
    return pl.kernel(
        body,
        out_type=[
            jax.ShapeDtypeStruct((E, H), _F32),
            jax.ShapeDtypeStruct((_NW * 8, 2 * H), _F32),
        ],
        mesh=_sc_mesh(),
        scratch_types=dbl + dbl + [
            pltpu.VMEM((H // _LN, _LN), _F32),
            pltpu.VMEM((H // _LN, _LN), _F32),
            pltpu.VMEM((8, 2 * H), _F32),
            pltpu.SemaphoreType.DMA,
            pltpu.SemaphoreType.DMA,
            pltpu.SemaphoreType.DMA,
            pltpu.SemaphoreType.DMA,
        ],
    )


def _p2_body(e_new, vh, src, dst, den_p, num_p,
             eidx0, sidx0, enb0, pay0, vrows0,
             eidx1, sidx1, enb1, pay1, vrows1,
             wbuf, acc, sin0, sin1, ssc0, ssc1,
             *, EW, B, H, N, WR):
    c = lax.axis_index("c")
    s = lax.axis_index("s")
    wid = s * _NC + c
    e0 = wid * EW
    nch = EW // B
    HS = H // _LN
    nrc = N // WR
    z = jnp.zeros((_LN,), _F32)

    slots = (
        (eidx0, sidx0, enb0, pay0, vrows0, sin0, ssc0),
        (eidx1, sidx1, enb1, pay1, vrows1, sin1, ssc1),
    )

    for d, outp in ((0, den_p), (1, num_p)):
        @pl.loop(0, WR)
        def _zb(i):
            for j in range(HS):
                wbuf[i, pl.ds(j * _LN, _LN)] = z

        @pl.loop(s, nrc, step=_NS)
        def _za(t):
            pltpu.sync_copy(wbuf, acc.at[pl.ds(t * WR, WR)])

        plsc.subcore_barrier()

        def prefetch(k, sl):
            _, si, enb, _, vrows, sin, _ = sl
            base = e0 + k * B
            pltpu.async_copy(e_new.at[pl.ds(base, B)], enb, sin)
            if d == 1:
                pltpu.sync_copy(src.at[pl.ds(base, B)], si)
                pltpu.async_copy(vh.at[si], vrows, sin)

        def process(k, sl, wait_sc):
            ei, si, enb, pay, vrows, sin, ssc = sl
            pltpu.make_async_copy(e_new.at[pl.ds(0, B)], enb, sin).wait()
            if d == 1:
                pltpu.make_async_copy(vh.at[si], vrows, sin).wait()
            if wait_sc:
                pltpu.make_async_copy(pay, acc.at[ei], ssc).wait()
            pltpu.sync_copy(dst.at[pl.ds(e0 + k * B, B)], ei)

            @pl.loop(0, B)
            def _edge(i):
                for j in range(HS):
                    sl2 = pl.ds(j * _LN, _LN)
                    xv = enb[i, sl2]
                    sg = 1.0 / (1.0 + jnp.exp(-xv))
                    if d == 1:
                        sg = sg * vrows[i, sl2]
                    pay[i, sl2] = sg

            pltpu.async_copy(pay, acc.at[ei], ssc, add=True)

        prefetch(0, slots[0])
        prefetch(1, slots[1])
        process(0, slots[0], wait_sc=False)
        prefetch(2, slots[0])
        process(1, slots[1], wait_sc=False)

        @pl.loop(1, nch // 2)
        def _pair(t):
            k = 2 * t
            prefetch(k + 1, slots[1])
            process(k, slots[0], wait_sc=True)
            pl.when(k + 2 <= nch - 1)(lambda: prefetch(k + 2, slots[0]))
            process(k + 1, slots[1], wait_sc=True)

        pltpu.make_async_copy(pay0, acc.at[eidx0], ssc0).wait()
        pltpu.make_async_copy(pay1, acc.at[eidx1], ssc1).wait()

        plsc.subcore_barrier()

        @pl.loop(s, nrc, step=_NS)
        def _wo(t):
            row = t * WR
            pltpu.sync_copy(acc.at[pl.ds(row, WR)], wbuf)
            pltpu.sync_copy(wbuf, outp.at[c, pl.ds(row, WR)])

        plsc.subcore_barrier()


def _make_p2(N, E, H):
    EW = E // _NW
    B = 40
    WR = 80
    assert N % WR == 0 and EW % B == 0 and (EW // B) % 2 == 0
    body = functools.partial(_p2_body, EW=EW, B=B, H=H, N=N, WR=WR)
    dbl = [
        pltpu.VMEM((B,), jnp.int32),
        pltpu.VMEM((B,), jnp.int32),
        pltpu.VMEM((B, H), _F32),
        pltpu.VMEM((B, H), _F32),
        pltpu.VMEM((B, H), _F32),
    ]
    return pl.kernel(
        body,
        out_type=[
            jax.ShapeDtypeStruct((_NC, N, H), _F32),
            jax.ShapeDtypeStruct((_NC, N, H), _F32),
        ],
        mesh=_sc_mesh(),
        scratch_types=dbl + dbl + [
            pltpu.VMEM((WR, H), _F32),
            pltpu.VMEM_SHARED((N, H), _F32),
            pltpu.SemaphoreType.DMA,
            pltpu.SemaphoreType.DMA,
            pltpu.SemaphoreType.DMA,
            pltpu.SemaphoreType.DMA,
        ],
    )


def _p3_body(p12m, p3m, src, dst, sv_out,
             sidx0, didx0, r10, r20, r30, ob0,
             sidx1, didx1, r11, r21, r31, ob1,
             sin0, sin1, sout0, sout1,
             *, EW, B, HS_DIM):
    c = lax.axis_index("c")
    s = lax.axis_index("s")
    wid = s * _NC + c
    e0 = wid * EW
    nch = EW // B
    JS = HS_DIM // _LN

    slots = (
        (sidx0, didx0, r10, r20, r30, ob0, sin0, sout0),
        (sidx1, didx1, r11, r21, r31, ob1, sin1, sout1),
    )

    def prefetch(k, sl):
        si, di, r1, r2, r3, _, sin, _ = sl
        base = e0 + k * B
        pltpu.sync_copy(src.at[pl.ds(base, B)], si)
        pltpu.sync_copy(dst.at[pl.ds(base, B)], di)
        pltpu.async_copy(p12m.at[si], r1, sin)
        pltpu.async_copy(p12m.at[di], r2, sin)
        pltpu.async_copy(p3m.at[pl.ds(base, B)], r3, sin)

    def process(k, sl, wait_out):
        si, di, r1, r2, r3, ob, sin, sout = sl
        pltpu.make_async_copy(p12m.at[si], r1, sin).wait()
        pltpu.make_async_copy(p12m.at[di], r2, sin).wait()
        pltpu.make_async_copy(p3m.at[pl.ds(0, B)], r3, sin).wait()
        if wait_out:
            pltpu.make_async_copy(ob, sv_out.at[pl.ds(0, B)], sout).wait()

        @pl.loop(0, B)
        def _edge(i):
            for j4 in range(JS):
                slx = pl.ds(j4 * _LN, _LN)
                v = (r1[i, slx] + r2[i, pl.ds(HS_DIM + j4 * _LN, _LN)]
                     + r3[i, slx])
                ob[i, slx] = jnp.maximum(v, 0.0)

        pltpu.async_copy(ob, sv_out.at[pl.ds(e0 + k * B, B)], sout)

    prefetch(0, slots[0])
    prefetch(1, slots[1])
    process(0, slots[0], wait_out=False)
    prefetch(2, slots[0])
    process(1, slots[1], wait_out=False)

    @pl.loop(1, nch // 2)
    def _pair(t):
        k = 2 * t
        prefetch(k + 1, slots[1])
        process(k, slots[0], wait_out=True)
        pl.when(k + 2 <= nch - 1)(lambda: prefetch(k + 2, slots[0]))
        process(k + 1, slots[1], wait_out=True)

    if nch % 2 == 1:
        process(nch - 1, slots[0], wait_out=True)

    pltpu.make_async_copy(ob0, sv_out.at[pl.ds(0, B)], sout0).wait()
    pltpu.make_async_copy(ob1, sv_out.at[pl.ds(0, B)], sout1).wait()


def _make_p3(E, HS_DIM):
    EW = E // _NW
    B = 80
    assert EW % B == 0 and EW // B >= 4
    body = functools.partial(_p3_body, EW=EW, B=B, HS_DIM=HS_DIM)
    dbl = [
        pltpu.VMEM((B,), jnp.int32),
        pltpu.VMEM((B,), jnp.int32),
        pltpu.VMEM((B, 2 * HS_DIM), _F32),
        pltpu.VMEM((B, 2 * HS_DIM), _F32),
        pltpu.VMEM((B, HS_DIM), _F32),
        pltpu.VMEM((B, HS_DIM), _F32),
    ]
    return pl.kernel(
        body,
        out_type=jax.ShapeDtypeStruct((E, HS_DIM), _F32),
        mesh=_sc_mesh(),
        scratch_types=dbl + dbl + [
            pltpu.SemaphoreType.DMA,
            pltpu.SemaphoreType.DMA,
            pltpu.SemaphoreType.DMA,
            pltpu.SemaphoreType.DMA,
        ],
    )


def _pe_body(pe_ref, wt_ref, b_ref, o_ref):
    o_ref[...] = (
        jnp.dot(pe_ref[...], wt_ref[...], preferred_element_type=_F32)
        + b_ref[...]
    )


def _edge_enc_body(e_ref, w1t, b1, w2t, b2, w3t, b3, o_ref, o_a3):
    t = jnp.dot(e_ref[...], w1t[...], preferred_element_type=_F32) + b1[...]
    t = jnp.maximum(t, 0.0)
    ee = jnp.dot(t, w2t[...], preferred_element_type=_F32) + b2[...]
    o_ref[...] = ee
    o_a3[...] = jnp.dot(ee, w3t[...], preferred_element_type=_F32) + b3[...]


def _node_mm_body(h_ref, w1t, b1, w2t, b2, w3t, b3, w4t, b4, o1, o2, o3, o4):
    h = h_ref[...]
    o1[...] = jnp.dot(h, w1t[...], preferred_element_type=_F32) + b1[...]
    o2[...] = jnp.dot(h, w2t[...], preferred_element_type=_F32) + b2[...]
    o3[...] = jnp.dot(h, w3t[...], preferred_element_type=_F32) + b3[...]
    o4[...] = jnp.dot(h, w4t[...], preferred_element_type=_F32) + b4[...]


def _mm_bias_body(x_ref, wt_ref, b_ref, o_ref):
    o_ref[...] = (
        jnp.dot(x_ref[...], wt_ref[...], preferred_element_type=_F32)
        + b_ref[...]
    )


def _bn_e_body(ehat, ee, stats, g, b, wt, bt, o, o_nxt, *, E, H):
    st = stats[...]
    ssum = jnp.sum(st[:, :H], axis=0)
    ssq = jnp.sum(st[:, H:], axis=0)
    mean = ssum * (1.0 / E)
    var = ssq * (1.0 / E) - mean * mean
    rstd = lax.rsqrt(var + 1e-5)
    xh = (ehat[...] - mean[None, :]) * rstd[None, :] * g[...] + b[...]
    en = ee[...] + jnp.maximum(xh, 0.0)
    o[...] = en
    o_nxt[...] = jnp.dot(en, wt[...], preferred_element_type=_F32) + bt[...]


def _h_upd_body(h, uh, denp, nump, g, b, o):
    den = denp[0] + denp[1] + 1e-6
    num = nump[0] + nump[1]
    t = uh[...] + num / den
    mean = jnp.mean(t, axis=0, keepdims=True)
    var = jnp.mean((t - mean) ** 2, axis=0, keepdims=True)
    xh = (t - mean) * lax.rsqrt(var + 1e-5) * g[...] + b[...]
    o[...] = h[...] + jnp.maximum(xh, 0.0)


def _full_spec(shape):
    return pl.BlockSpec(shape, lambda i: (0,) * len(shape))


def _row_spec(be, cols):
    return pl.BlockSpec((be, cols), lambda i: (i, 0))


def kernel(x, e, pe, edge_index, params):
    p = params
    N, H = x.shape
    E, D_EDGE = e.shape
    PE_DIM = pe.shape[1]
    H_EDGE = p["e1_w"].shape[0]
    HS_DIM = p["s1_w"].shape[0]
    L = p["A1_w"].shape[0]
    assert E % _NW == 0 and (E // _NW) % 80 == 0 and N % _NS == 0

    src = edge_index[0]
    dst = edge_index[1]

    BE = 4000
    grid_e = (E // BE,)

    h = pl.pallas_call(
        _pe_body,
        out_shape=jax.ShapeDtypeStruct((N, H), _F32),
    )(pe, p["pe_w"].T, p["pe_b"][None, :])

    ee, a3e = pl.pallas_call(
        _edge_enc_body,
        grid=grid_e,
        in_specs=[
            _row_spec(BE, D_EDGE),
            _full_spec((D_EDGE, H_EDGE)),
            _full_spec((1, H_EDGE)),
            _full_spec((H_EDGE, H)),
            _full_spec((1, H)),
            _full_spec((H, H)),
            _full_spec((1, H)),
        ],
        out_specs=[_row_spec(BE, H), _row_spec(BE, H)],
        out_shape=[jax.ShapeDtypeStruct((E, H), _F32)] * 2,
    )(e, p["e1_w"].T, p["e1_b"][None, :], p["e2_w"].T, p["e2_b"][None, :],
      p["A3_w"][0].T, p["A3_b"][0][None, :])

    p1_fn = _make_p1(N, E, H)
    p2_fn = _make_p2(N, E, H)
    s1 = p["s1_w"]

    for l in range(L):
        a1h, a2h, uh, vh = pl.pallas_call(
            _node_mm_body,
            out_shape=[jax.ShapeDtypeStruct((N, H), _F32)] * 4,
        )(
            h,
            p["A1_w"][l].T, p["A1_b"][l][None, :],
            p["A2_w"][l].T, p["A2_b"][l][None, :],
            p["U_w"][l].T, p["U_b"][l][None, :],
            p["V_w"][l].T, p["V_b"][l][None, :],
        )

        ehat, stats = p1_fn(a1h, a2h, a3e, src, dst)

        if l < L - 1:
            wt, bt = p["A3_w"][l + 1].T, p["A3_b"][l + 1][None, :]
        else:
            wt, bt = s1[:, 2 * H:].T, p["s1_b"][None, :]
        nxt_w = wt.shape[1]
        e_new, nxt = pl.pallas_call(
            functools.partial(_bn_e_body, E=E, H=H),
            grid=grid_e,
            in_specs=[
                _row_spec(BE, H),
                _row_spec(BE, H),
                _full_spec((_NW * 8, 2 * H)),
                _full_spec((1, H)),
                _full_spec((1, H)),
                _full_spec((H, nxt_w)),
                _full_spec((1, nxt_w)),
            ],
            out_specs=[_row_spec(BE, H), _row_spec(BE, nxt_w)],
            out_shape=[
                jax.ShapeDtypeStruct((E, H), _F32),
                jax.ShapeDtypeStruct((E, nxt_w), _F32),
            ],
        )(ehat, ee, stats, p["bn_e_g"][l][None, :], p["bn_e_b"][l][None, :],
          wt, bt)

        den_p, num_p = p2_fn(e_new, vh, src, dst)

        h = pl.pallas_call(
            _h_upd_body,
            out_shape=jax.ShapeDtypeStruct((N, H), _F32),
        )(h, uh, den_p, num_p, p["bn_h_g"][l][None, :], p["bn_h_b"][l][None, :])

        ee = e_new
        a3e = nxt

    p3s = a3e
    wcat = jnp.concatenate([s1[:, :H].T, s1[:, H:2 * H].T], axis=1)
    p12 = pl.pallas_call(
        _mm_bias_body,
        out_shape=jax.ShapeDtypeStruct((N, 2 * HS_DIM), _F32),
    )(h, wcat, jnp.zeros((1, 2 * HS_DIM), _F32))

    sv = _make_p3(E, HS_DIM)(p12, p3s, src, dst)

    scores = pl.pallas_call(
        _mm_bias_body,
        grid=grid_e,
        in_specs=[
            _row_spec(BE, HS_DIM),
            _full_spec((HS_DIM, 1)),
            _full_spec((1, 1)),
        ],
        out_specs=_row_spec(BE, 1),
        out_shape=jax.ShapeDtypeStruct((E, 1), _F32),
    )(sv, p["s2_w"].T, p["s2_b"][None, :])
    return scores
